# baseline (speedup 1.0000x reference)
import jax
import jax.numpy as jnp
from jax import lax
from jax.experimental import pallas as pl
from jax.experimental.pallas import tpu as pltpu
from jax.experimental.pallas import tpu_sc as plsc

B = 10000
K = 32
S = 16
D = 128
E = 64
L = 16
NC = 2
NS = 16
NW = NC * NS
G = 8
CHUNKS = (5000, 5000)


def _select16(bs_v, ns_v, nb_v, row, outs_b, idxn_b, j):
    iota = lax.iota(jnp.int32, L)
    rfull = jnp.full((L,), row, jnp.int32)
    center = plsc.load_gather(bs_v, [rfull])
    sa = plsc.load_gather(ns_v, [iota, rfull])
    sb = plsc.load_gather(ns_v, [iota + L, rfull])
    da = jnp.abs(sa - center)
    db = jnp.abs(sb - center)
    ia = plsc.load_gather(nb_v, [iota, rfull])
    ib = plsc.load_gather(nb_v, [iota + L, rfull])
    ka, va = plsc.sort_key_val(da, ia)
    kb, vb = plsc.sort_key_val(db, ib)
    rkb = lax.rev(kb, (0,))
    rvb = lax.rev(vb, (0,))
    ta = ka <= rkb
    km = jnp.where(ta, ka, rkb)
    vm = jnp.where(ta, va, rvb)
    ks, vs = plsc.sort_key_val(km, vm)
    plsc.store_scatter(outs_b, [iota, rfull], ks)
    idxn_b[pl.ds(j * L, L)] = vs


def _make_sc_stage(bn):
    ngt = bn // G
    c0 = ngt // NW
    thr = NW - (ngt - NW * c0)
    cst = (c0 + 1) * G

    def body(bs_hbm, ns_hbm, nb_hbm, nd_hbm, feat_hbm, self_hbm, agg_hbm, samp_hbm,
             bs_v, ns_v, nb_v, nd_v, selfr_v,
             idxn0, idxn1,
             rown0, rown1,
             outa0, outa1, outst,
             gsem0, gsem1, osem0, osem1, ssem):
        wid = lax.axis_index("s") * NC + lax.axis_index("c")
        base = G * (c0 * wid + jnp.maximum(wid - thr, 0))
        ngroups = c0 + jnp.where(wid >= thr, 1, 0)

        pltpu.async_copy(bs_hbm.at[pl.ds(base, cst)], bs_v, ssem)
        pltpu.async_copy(ns_hbm.at[pl.ds(0, K), pl.ds(base, cst)], ns_v, ssem)
        pltpu.async_copy(nb_hbm.at[pl.ds(0, K), pl.ds(base, cst)], nb_v, ssem)
        pltpu.async_copy(nd_hbm.at[pl.ds(base, cst)], nd_v, ssem)
        pltpu.make_async_copy(bs_hbm.at[pl.ds(base, cst)], bs_v, ssem).wait()
        pltpu.make_async_copy(ns_hbm.at[pl.ds(0, K), pl.ds(base, cst)], ns_v, ssem).wait()
        pltpu.make_async_copy(nb_hbm.at[pl.ds(0, K), pl.ds(base, cst)], nb_v, ssem).wait()
        pltpu.make_async_copy(nd_hbm.at[pl.ds(base, cst)], nd_v, ssem).wait()

        sparts = [(o, min(128, cst - o)) for o in range(0, cst, 128)]
        for off, sz in sparts:
            pltpu.async_copy(feat_hbm.at[nd_v.at[pl.ds(off, sz)]],
                             selfr_v.at[pl.ds(off, sz)], ssem)

        idxn = (idxn0, idxn1)
        rown = (rown0, rown1)
        outa = (outa0, outa1)
        gsem = (gsem0, gsem1)
        osem = (osem0, osem1)

        def select_group(g, b):
            row0 = g * G
            for j in range(G):
                _select16(bs_v, ns_v, nb_v, row0 + j, outst, idxn[b], j)

        def accumulate_group(gp, b):
            rn = rown[b]
            oa = outa[b]

            @plsc.parallel_loop(0, G * (D // L), 1, unroll=4)
            def _(i):
                j = i >> 3
                d = i & 7
                r0 = j * S
                sl = pl.ds(d * L, L)
                p = [rn[r0 + s, sl] for s in range(4)]
                for s in range(4, S):
                    p[s % 4] = p[s % 4] + rn[r0 + s, sl]
                oa[j, sl] = (p[0] + p[1]) + (p[2] + p[3])

            dst = base + gp * G
            pltpu.async_copy(outa[b], agg_hbm.at[pl.ds(dst, G)], osem[b])

        def step(g, b):
            @pl.when((g >= 2) & (g <= ngroups))
            def _():
                pltpu.make_async_copy(outa[b], agg_hbm.at[pl.ds(0, G)], osem[b]).wait()

            @pl.when(g < ngroups)
            def _():
                select_group(g, b)
                pltpu.async_copy(feat_hbm.at[idxn[b]], rown[b], gsem[b])

            @pl.when((g >= 1) & (g <= ngroups))
            def _():
                nb = 1 - b
                pltpu.make_async_copy(feat_hbm.at[idxn[nb]], rown[nb], gsem[nb]).wait()
                accumulate_group(g - 1, nb)

            @pl.when(g == 16)
            def _():
                for off, sz in sparts:
                    pltpu.make_async_copy(feat_hbm.at[nd_v.at[pl.ds(off, sz)]],
                                          selfr_v.at[pl.ds(off, sz)], ssem).wait()
                pltpu.async_copy(selfr_v.at[pl.ds(0, c0 * G)],
                                 self_hbm.at[pl.ds(base, c0 * G)], ssem)

        def outer(t, carry):
            step(2 * t, 0)
            step(2 * t + 1, 1)
            return carry

        lax.fori_loop(0, (ngroups + 2) // 2, outer, 0)

        @pl.when(((ngroups - 1) % 2) == 0)
        def _():
            pltpu.make_async_copy(outa[0], agg_hbm.at[pl.ds(0, G)], osem[0]).wait()

        @pl.when(((ngroups - 1) % 2) == 1)
        def _():
            pltpu.make_async_copy(outa[1], agg_hbm.at[pl.ds(0, G)], osem[1]).wait()

        pltpu.sync_copy(outst.at[pl.ds(0, S), pl.ds(0, c0 * G)],
                        samp_hbm.at[pl.ds(0, S), pl.ds(base, c0 * G)])

        @pl.when(wid >= thr)
        def _():
            pltpu.sync_copy(outst.at[pl.ds(0, S), pl.ds(c0 * G, G)],
                            samp_hbm.at[pl.ds(0, S), pl.ds(base + c0 * G, G)])

        pltpu.make_async_copy(selfr_v.at[pl.ds(0, c0 * G)],
                              self_hbm.at[pl.ds(base, c0 * G)], ssem).wait()

        @pl.when(wid >= thr)
        def _():
            pltpu.sync_copy(selfr_v.at[pl.ds(c0 * G, G)],
                            self_hbm.at[pl.ds(base + c0 * G, G)])

    mesh = plsc.VectorSubcoreMesh(core_axis_name="c", subcore_axis_name="s")
    return pl.kernel(
        body,
        out_type=(
            jax.ShapeDtypeStruct((bn, D), jnp.float32),
            jax.ShapeDtypeStruct((bn, D), jnp.float32),
            jax.ShapeDtypeStruct((S, bn), jnp.float32),
        ),
        mesh=mesh,
        compiler_params=pltpu.CompilerParams(needs_layout_passes=False,
                                             use_tc_tiling_on_sc=False),
        scratch_types=[
            pltpu.VMEM((cst,), jnp.float32),
            pltpu.VMEM((K, cst), jnp.float32),
            pltpu.VMEM((K, cst), jnp.int32),
            pltpu.VMEM((cst,), jnp.int32),
            pltpu.VMEM((cst, D), jnp.float32),
            pltpu.VMEM((G * S,), jnp.int32),
            pltpu.VMEM((G * S,), jnp.int32),
            pltpu.VMEM((G * S, D), jnp.float32),
            pltpu.VMEM((G * S, D), jnp.float32),
            pltpu.VMEM((G, D), jnp.float32),
            pltpu.VMEM((G, D), jnp.float32),
            pltpu.VMEM((S, cst), jnp.float32),
            pltpu.SemaphoreType.DMA,
            pltpu.SemaphoreType.DMA,
            pltpu.SemaphoreType.DMA,
            pltpu.SemaphoreType.DMA,
            pltpu.SemaphoreType.DMA,
        ],
    )


_SC_STAGES = {bn: _make_sc_stage(bn) for bn in set(CHUNKS)}


def _mm_body(sf_ref, ag_ref, wt_ref, wb_ref, out_ref):
    dn = (((0,), (1,)), ((), ()))
    acc = lax.dot_general(wt_ref[...], sf_ref[...], dn,
                          preferred_element_type=jnp.float32)
    acc += lax.dot_general(wb_ref[...], ag_ref[...], dn,
                           preferred_element_type=jnp.float32)
    out_ref[...] = jnp.maximum(acc, 0.0)


def _tc_matmul(sf, ag, wt, wb):
    bn = sf.shape[0]
    return pl.pallas_call(
        _mm_body,
        out_shape=jax.ShapeDtypeStruct((E, bn), jnp.float32),
    )(sf, ag, wt, wb)


def kernel(nodes, neighs, batch_scores, neigh_scores, features, weight, num_sample):
    ns0t = neigh_scores[:, :, 0].T
    nb0t = neighs.astype(jnp.int32).T
    bs0 = batch_scores[:, 0]
    ndi = nodes.astype(jnp.int32)
    inv = 1.0 / jnp.asarray(num_sample, jnp.float32)
    wt = weight[:D]
    wb = weight[D:] * inv

    tos, samps = [], []
    off = 0
    for bn in CHUNKS:
        sl = slice(off, off + bn)
        off += bn
        sf, ag, samp = _SC_STAGES[bn](bs0[sl], ns0t[:, sl], nb0t[:, sl],
                                      ndi[sl], features)
        sf = sf.reshape(bn * D).reshape(bn, D)
        ag = ag.reshape(bn * D).reshape(bn, D)
        tos.append(_tc_matmul(sf, ag, wt, wb))
        samps.append(samp)
    to_feats = jnp.concatenate(tos, axis=1).T
    samp_scores = jnp.concatenate(samps, axis=1).T
    return (to_feats, samp_scores)


# speedup vs baseline: 1.1636x; 1.1636x over previous
import jax
import jax.numpy as jnp
from jax import lax
from jax.experimental import pallas as pl
from jax.experimental.pallas import tpu as pltpu
from jax.experimental.pallas import tpu_sc as plsc

B = 10000
K = 32
S = 16
D = 128
E = 64
L = 16
NC = 2
NS = 16
NW = NC * NS
G = 8
CHUNKS = (10000,)


def _select16(bs_v, ns_v, nb_v, row, outs_b, idxn_b, j):
    iota = lax.iota(jnp.int32, L)
    rfull = jnp.full((L,), row, jnp.int32)
    center = plsc.load_gather(bs_v, [rfull])
    sa = plsc.load_gather(ns_v, [iota, rfull])
    sb = plsc.load_gather(ns_v, [iota + L, rfull])
    da = jnp.abs(sa - center)
    db = jnp.abs(sb - center)
    ia = plsc.load_gather(nb_v, [iota, rfull])
    ib = plsc.load_gather(nb_v, [iota + L, rfull])
    ka, va = plsc.sort_key_val(da, ia)
    kb, vb = plsc.sort_key_val(db, ib)
    rkb = lax.rev(kb, (0,))
    rvb = lax.rev(vb, (0,))
    ta = ka <= rkb
    km = jnp.where(ta, ka, rkb)
    vm = jnp.where(ta, va, rvb)
    ks, vs = plsc.sort_key_val(km, vm)
    plsc.store_scatter(outs_b, [iota, rfull], ks)
    idxn_b[pl.ds(j * L, L)] = vs


def _make_sc_stage(bn):
    ngt = bn // G
    c0 = ngt // NW
    thr = NW - (ngt - NW * c0)
    cst = (c0 + 1) * G

    def body(bs_hbm, ns_hbm, nb_hbm, nd_hbm, feat_hbm, self_hbm, agg_hbm, samp_hbm,
             bs_v, ns_v, nb_v, nd_v, selfr_v,
             idxn0, idxn1,
             rown0, rown1,
             outa0, outa1, outst,
             gsem0, gsem1, osem0, osem1, ssem):
        wid = lax.axis_index("s") * NC + lax.axis_index("c")
        base = G * (c0 * wid + jnp.maximum(wid - thr, 0))
        ngroups = c0 + jnp.where(wid >= thr, 1, 0)

        pltpu.async_copy(bs_hbm.at[pl.ds(base, cst)], bs_v, ssem)
        pltpu.async_copy(ns_hbm.at[pl.ds(0, K), pl.ds(base, cst)], ns_v, ssem)
        pltpu.async_copy(nb_hbm.at[pl.ds(0, K), pl.ds(base, cst)], nb_v, ssem)
        pltpu.async_copy(nd_hbm.at[pl.ds(base, cst)], nd_v, ssem)
        pltpu.make_async_copy(bs_hbm.at[pl.ds(base, cst)], bs_v, ssem).wait()
        pltpu.make_async_copy(ns_hbm.at[pl.ds(0, K), pl.ds(base, cst)], ns_v, ssem).wait()
        pltpu.make_async_copy(nb_hbm.at[pl.ds(0, K), pl.ds(base, cst)], nb_v, ssem).wait()
        pltpu.make_async_copy(nd_hbm.at[pl.ds(base, cst)], nd_v, ssem).wait()

        sparts = [(o, min(128, cst - o)) for o in range(0, cst, 128)]
        for off, sz in sparts:
            pltpu.async_copy(feat_hbm.at[nd_v.at[pl.ds(off, sz)]],
                             selfr_v.at[pl.ds(off, sz)], ssem)

        idxn = (idxn0, idxn1)
        rown = (rown0, rown1)
        outa = (outa0, outa1)
        gsem = (gsem0, gsem1)
        osem = (osem0, osem1)

        def select_group(g, b):
            row0 = g * G
            for j in range(G):
                _select16(bs_v, ns_v, nb_v, row0 + j, outst, idxn[b], j)

        def accumulate_group(gp, b):
            rn = rown[b]
            oa = outa[b]

            @plsc.parallel_loop(0, G * (D // L), 1, unroll=4)
            def _(i):
                j = i >> 3
                d = i & 7
                r0 = j * S
                sl = pl.ds(d * L, L)
                p = [rn[r0 + s, sl] for s in range(4)]
                for s in range(4, S):
                    p[s % 4] = p[s % 4] + rn[r0 + s, sl]
                oa[j, sl] = (p[0] + p[1]) + (p[2] + p[3])

            dst = base + gp * G
            pltpu.async_copy(outa[b], agg_hbm.at[pl.ds(dst, G)], osem[b])

        def step(g, b):
            @pl.when((g >= 2) & (g <= ngroups))
            def _():
                pltpu.make_async_copy(outa[b], agg_hbm.at[pl.ds(0, G)], osem[b]).wait()

            @pl.when(g < ngroups)
            def _():
                select_group(g, b)
                pltpu.async_copy(feat_hbm.at[idxn[b]], rown[b], gsem[b])

            @pl.when((g >= 1) & (g <= ngroups))
            def _():
                nb = 1 - b
                pltpu.make_async_copy(feat_hbm.at[idxn[nb]], rown[nb], gsem[nb]).wait()
                accumulate_group(g - 1, nb)

            @pl.when(g == 16)
            def _():
                for off, sz in sparts:
                    pltpu.make_async_copy(feat_hbm.at[nd_v.at[pl.ds(off, sz)]],
                                          selfr_v.at[pl.ds(off, sz)], ssem).wait()
                pltpu.async_copy(selfr_v.at[pl.ds(0, c0 * G)],
                                 self_hbm.at[pl.ds(base, c0 * G)], ssem)

        def outer(t, carry):
            step(2 * t, 0)
            step(2 * t + 1, 1)
            return carry

        lax.fori_loop(0, (ngroups + 2) // 2, outer, 0)

        @pl.when(((ngroups - 1) % 2) == 0)
        def _():
            pltpu.make_async_copy(outa[0], agg_hbm.at[pl.ds(0, G)], osem[0]).wait()

        @pl.when(((ngroups - 1) % 2) == 1)
        def _():
            pltpu.make_async_copy(outa[1], agg_hbm.at[pl.ds(0, G)], osem[1]).wait()

        pltpu.sync_copy(outst.at[pl.ds(0, S), pl.ds(0, c0 * G)],
                        samp_hbm.at[pl.ds(0, S), pl.ds(base, c0 * G)])

        @pl.when(wid >= thr)
        def _():
            pltpu.sync_copy(outst.at[pl.ds(0, S), pl.ds(c0 * G, G)],
                            samp_hbm.at[pl.ds(0, S), pl.ds(base + c0 * G, G)])

        pltpu.make_async_copy(selfr_v.at[pl.ds(0, c0 * G)],
                              self_hbm.at[pl.ds(base, c0 * G)], ssem).wait()

        @pl.when(wid >= thr)
        def _():
            pltpu.sync_copy(selfr_v.at[pl.ds(c0 * G, G)],
                            self_hbm.at[pl.ds(base + c0 * G, G)])

    mesh = plsc.VectorSubcoreMesh(core_axis_name="c", subcore_axis_name="s")
    return pl.kernel(
        body,
        out_type=(
            jax.ShapeDtypeStruct((bn, D), jnp.float32),
            jax.ShapeDtypeStruct((bn, D), jnp.float32),
            jax.ShapeDtypeStruct((S, bn), jnp.float32),
        ),
        mesh=mesh,
        compiler_params=pltpu.CompilerParams(needs_layout_passes=False,
                                             use_tc_tiling_on_sc=False),
        scratch_types=[
            pltpu.VMEM((cst,), jnp.float32),
            pltpu.VMEM((K, cst), jnp.float32),
            pltpu.VMEM((K, cst), jnp.int32),
            pltpu.VMEM((cst,), jnp.int32),
            pltpu.VMEM((cst, D), jnp.float32),
            pltpu.VMEM((G * S,), jnp.int32),
            pltpu.VMEM((G * S,), jnp.int32),
            pltpu.VMEM((G * S, D), jnp.float32),
            pltpu.VMEM((G * S, D), jnp.float32),
            pltpu.VMEM((G, D), jnp.float32),
            pltpu.VMEM((G, D), jnp.float32),
            pltpu.VMEM((S, cst), jnp.float32),
            pltpu.SemaphoreType.DMA,
            pltpu.SemaphoreType.DMA,
            pltpu.SemaphoreType.DMA,
            pltpu.SemaphoreType.DMA,
            pltpu.SemaphoreType.DMA,
        ],
    )


_SC_STAGES = {bn: _make_sc_stage(bn) for bn in set(CHUNKS)}


def _mm_body(sf_ref, ag_ref, wt_ref, wb_ref, out_ref):
    dn = (((0,), (1,)), ((), ()))
    acc = lax.dot_general(wt_ref[...], sf_ref[...], dn,
                          preferred_element_type=jnp.float32)
    acc += lax.dot_general(wb_ref[...], ag_ref[...], dn,
                           preferred_element_type=jnp.float32)
    out_ref[...] = jnp.maximum(acc, 0.0)


def _tc_matmul(sf, ag, wt, wb):
    bn = sf.shape[0]
    return pl.pallas_call(
        _mm_body,
        out_shape=jax.ShapeDtypeStruct((E, bn), jnp.float32),
    )(sf, ag, wt, wb)


def kernel(nodes, neighs, batch_scores, neigh_scores, features, weight, num_sample):
    ns0t = neigh_scores[:, :, 0].T
    nb0t = neighs.astype(jnp.int32).T
    bs0 = batch_scores[:, 0]
    ndi = nodes.astype(jnp.int32)
    inv = 1.0 / jnp.asarray(num_sample, jnp.float32)
    wt = weight[:D]
    wb = weight[D:] * inv

    tos, samps = [], []
    off = 0
    for bn in CHUNKS:
        sl = slice(off, off + bn)
        off += bn
        sf, ag, samp = _SC_STAGES[bn](bs0[sl], ns0t[:, sl], nb0t[:, sl],
                                      ndi[sl], features)
        sf = sf.reshape(bn * D).reshape(bn, D)
        ag = ag.reshape(bn * D).reshape(bn, D)
        tos.append(_tc_matmul(sf, ag, wt, wb))
        samps.append(samp)
    to_feats = jnp.concatenate(tos, axis=1).T
    samp_scores = jnp.concatenate(samps, axis=1).T
    return (to_feats, samp_scores)


# speedup vs baseline: 1.1646x; 1.0008x over previous
import jax
import jax.numpy as jnp
from jax import lax
from jax.experimental import pallas as pl
from jax.experimental.pallas import tpu as pltpu
from jax.experimental.pallas import tpu_sc as plsc

B = 10000
K = 32
S = 16
D = 128
E = 64
L = 16
NC = 2
NS = 16
NW = NC * NS
G = 8
CHUNKS = (10000,)


def _select16(bs_v, ns_v, nb_v, row, outs_b, idxn_b, j):
    iota = lax.iota(jnp.int32, L)
    rfull = jnp.full((L,), row, jnp.int32)
    center = plsc.load_gather(bs_v, [rfull])
    sa = plsc.load_gather(ns_v, [iota, rfull])
    sb = plsc.load_gather(ns_v, [iota + L, rfull])
    da = jnp.abs(sa - center)
    db = jnp.abs(sb - center)
    ia = plsc.load_gather(nb_v, [iota, rfull])
    ib = plsc.load_gather(nb_v, [iota + L, rfull])
    ka, va = plsc.sort_key_val(da, ia)
    kb, vb = plsc.sort_key_val(db, ib)
    rkb = lax.rev(kb, (0,))
    rvb = lax.rev(vb, (0,))
    ta = ka <= rkb
    km = jnp.where(ta, ka, rkb)
    vm = jnp.where(ta, va, rvb)
    ks, vs = plsc.sort_key_val(km, vm)
    plsc.store_scatter(outs_b, [iota, rfull], ks)
    idxn_b[pl.ds(j * L, L)] = vs


def _make_sc_stage(bn):
    ngt = bn // G
    c0 = ngt // NW
    thr = NW - (ngt - NW * c0)
    cst = (c0 + 1) * G
    bp = ((bn + 127) // 128) * 128

    def body(bs_hbm, ns_hbm, nb_hbm, nd_hbm, feat_hbm, self_hbm, agg_hbm, samp_hbm,
             bs_v, ns_v, nb_v, nd_v, selfr_v,
             idxn0, idxn1,
             rown0, rown1,
             outa0, outa1, outst,
             gsem0, gsem1, osem0, osem1, ssem):
        wid = lax.axis_index("s") * NC + lax.axis_index("c")
        base = G * (c0 * wid + jnp.maximum(wid - thr, 0))
        ngroups = c0 + jnp.where(wid >= thr, 1, 0)

        pltpu.async_copy(bs_hbm.at[pl.ds(base, cst)], bs_v, ssem)
        pltpu.async_copy(ns_hbm.at[pl.ds(0, K), pl.ds(base, cst)], ns_v, ssem)
        pltpu.async_copy(nb_hbm.at[pl.ds(0, K), pl.ds(base, cst)], nb_v, ssem)
        pltpu.async_copy(nd_hbm.at[pl.ds(base, cst)], nd_v, ssem)
        pltpu.make_async_copy(bs_hbm.at[pl.ds(base, cst)], bs_v, ssem).wait()
        pltpu.make_async_copy(ns_hbm.at[pl.ds(0, K), pl.ds(base, cst)], ns_v, ssem).wait()
        pltpu.make_async_copy(nb_hbm.at[pl.ds(0, K), pl.ds(base, cst)], nb_v, ssem).wait()
        pltpu.make_async_copy(nd_hbm.at[pl.ds(base, cst)], nd_v, ssem).wait()

        sparts = [(o, min(128, cst - o)) for o in range(0, cst, 128)]
        for off, sz in sparts:
            pltpu.async_copy(feat_hbm.at[nd_v.at[pl.ds(off, sz)]],
                             selfr_v.at[pl.ds(off, sz)], ssem)

        idxn = (idxn0, idxn1)
        rown = (rown0, rown1)
        outa = (outa0, outa1)
        gsem = (gsem0, gsem1)
        osem = (osem0, osem1)

        def select_group(g, b):
            row0 = g * G
            for j in range(G):
                _select16(bs_v, ns_v, nb_v, row0 + j, outst, idxn[b], j)

        def accumulate_group(gp, b):
            rn = rown[b]
            oa = outa[b]

            @plsc.parallel_loop(0, G * (D // L), 1, unroll=4)
            def _(i):
                j = i >> 3
                d = i & 7
                r0 = j * S
                sl = pl.ds(d * L, L)
                p = [rn[r0 + s, sl] for s in range(4)]
                for s in range(4, S):
                    p[s % 4] = p[s % 4] + rn[r0 + s, sl]
                oa[j, sl] = (p[0] + p[1]) + (p[2] + p[3])

            dst = base + gp * G
            pltpu.async_copy(outa[b], agg_hbm.at[pl.ds(dst, G)], osem[b])

        def step(g, b):
            @pl.when((g >= 2) & (g <= ngroups))
            def _():
                pltpu.make_async_copy(outa[b], agg_hbm.at[pl.ds(0, G)], osem[b]).wait()

            @pl.when(g < ngroups)
            def _():
                select_group(g, b)
                pltpu.async_copy(feat_hbm.at[idxn[b]], rown[b], gsem[b])

            @pl.when((g >= 1) & (g <= ngroups))
            def _():
                nb = 1 - b
                pltpu.make_async_copy(feat_hbm.at[idxn[nb]], rown[nb], gsem[nb]).wait()
                accumulate_group(g - 1, nb)

            @pl.when(g == 16)
            def _():
                for off, sz in sparts:
                    pltpu.make_async_copy(feat_hbm.at[nd_v.at[pl.ds(off, sz)]],
                                          selfr_v.at[pl.ds(off, sz)], ssem).wait()
                pltpu.async_copy(selfr_v.at[pl.ds(0, c0 * G)],
                                 self_hbm.at[pl.ds(base, c0 * G)], ssem)

        def outer(t, carry):
            step(2 * t, 0)
            step(2 * t + 1, 1)
            return carry

        lax.fori_loop(0, (ngroups + 2) // 2, outer, 0)

        @pl.when(((ngroups - 1) % 2) == 0)
        def _():
            pltpu.make_async_copy(outa[0], agg_hbm.at[pl.ds(0, G)], osem[0]).wait()

        @pl.when(((ngroups - 1) % 2) == 1)
        def _():
            pltpu.make_async_copy(outa[1], agg_hbm.at[pl.ds(0, G)], osem[1]).wait()

        pltpu.sync_copy(outst.at[pl.ds(0, S), pl.ds(0, c0 * G)],
                        samp_hbm.at[pl.ds(0, S), pl.ds(base, c0 * G)])

        @pl.when(wid >= thr)
        def _():
            pltpu.sync_copy(outst.at[pl.ds(0, S), pl.ds(c0 * G, G)],
                            samp_hbm.at[pl.ds(0, S), pl.ds(base + c0 * G, G)])

        pltpu.make_async_copy(selfr_v.at[pl.ds(0, c0 * G)],
                              self_hbm.at[pl.ds(base, c0 * G)], ssem).wait()

        @pl.when(wid >= thr)
        def _():
            pltpu.sync_copy(selfr_v.at[pl.ds(c0 * G, G)],
                            self_hbm.at[pl.ds(base + c0 * G, G)])

    mesh = plsc.VectorSubcoreMesh(core_axis_name="c", subcore_axis_name="s")
    return pl.kernel(
        body,
        out_type=(
            jax.ShapeDtypeStruct((bn, D), jnp.float32),
            jax.ShapeDtypeStruct((bn, D), jnp.float32),
            jax.ShapeDtypeStruct((S, bp), jnp.float32),
        ),
        mesh=mesh,
        compiler_params=pltpu.CompilerParams(needs_layout_passes=False,
                                             use_tc_tiling_on_sc=False),
        scratch_types=[
            pltpu.VMEM((cst,), jnp.float32),
            pltpu.VMEM((K, cst), jnp.float32),
            pltpu.VMEM((K, cst), jnp.int32),
            pltpu.VMEM((cst,), jnp.int32),
            pltpu.VMEM((cst, D), jnp.float32),
            pltpu.VMEM((G * S,), jnp.int32),
            pltpu.VMEM((G * S,), jnp.int32),
            pltpu.VMEM((G * S, D), jnp.float32),
            pltpu.VMEM((G * S, D), jnp.float32),
            pltpu.VMEM((G, D), jnp.float32),
            pltpu.VMEM((G, D), jnp.float32),
            pltpu.VMEM((S, cst), jnp.float32),
            pltpu.SemaphoreType.DMA,
            pltpu.SemaphoreType.DMA,
            pltpu.SemaphoreType.DMA,
            pltpu.SemaphoreType.DMA,
            pltpu.SemaphoreType.DMA,
        ],
    )


_SC_STAGES = {bn: _make_sc_stage(bn) for bn in set(CHUNKS)}


def _mm_body(sf_ref, ag_ref, wt_ref, wb_ref, out_ref):
    dn = (((0,), (1,)), ((), ()))
    acc = lax.dot_general(wt_ref[...], sf_ref[...], dn,
                          preferred_element_type=jnp.float32)
    acc += lax.dot_general(wb_ref[...], ag_ref[...], dn,
                           preferred_element_type=jnp.float32)
    out_ref[...] = jnp.maximum(acc, 0.0)


def _tc_matmul(sf, ag, wt, wb):
    bn = sf.shape[0]
    return pl.pallas_call(
        _mm_body,
        out_shape=jax.ShapeDtypeStruct((E, bn), jnp.float32),
    )(sf, ag, wt, wb)


def kernel(nodes, neighs, batch_scores, neigh_scores, features, weight, num_sample):
    ns0t = neigh_scores[:, :, 0].T
    nb0t = neighs.astype(jnp.int32).T
    bs0 = batch_scores[:, 0]
    ndi = nodes.astype(jnp.int32)
    inv = 1.0 / jnp.asarray(num_sample, jnp.float32)
    wt = weight[:D]
    wb = weight[D:] * inv

    tos, samps = [], []
    off = 0
    for bn in CHUNKS:
        sl = slice(off, off + bn)
        off += bn
        sf, ag, samp = _SC_STAGES[bn](bs0[sl], ns0t[:, sl], nb0t[:, sl],
                                      ndi[sl], features)
        sf = sf.reshape(bn * D).reshape(bn, D)
        ag = ag.reshape(bn * D).reshape(bn, D)
        bp = samp.shape[1]
        samp = samp.reshape(S * bp).reshape(S, bp)[:, :bn]
        tos.append(_tc_matmul(sf, ag, wt, wb))
        samps.append(samp)
    to_feats = jnp.concatenate(tos, axis=1).T
    samp_scores = jnp.concatenate(samps, axis=1).T
    return (to_feats, samp_scores)
